# Initial kernel scaffold; baseline (speedup 1.0000x reference)
import jax
import jax.numpy as jnp
from jax.experimental import pallas as pl


def kernel(features, edge_index, row_neg, col_neg, W1, b1, W2, b2):
    raise NotImplementedError("write your pallas kernel here")


# speedup vs baseline: 17.5818x; 17.5818x over previous
import jax
import jax.numpy as jnp
from jax import lax
from jax.experimental import pallas as pl
from jax.experimental.pallas import tpu as pltpu
from jax.experimental.pallas import tpu_sc as plsc

ALPHA = 0.0813
BETA = 0.947

NC, NS, L = 2, 16, 16
NW = NC * NS

N = 10000
E = 320000
D = 128
NP = 10240
RPT = NP // NS
B1 = 80
C1 = (E // NW) // B1
C2 = (E // NS) // B1
CP = (2 * E // NW) // B1


def _zero_fill(ref, nwords):
    def body(i, c):
        ref[pl.ds(i * L, L)] = jnp.zeros((L,), jnp.float32)
        return c
    lax.fori_loop(0, nwords // L, body, 0)


def _deg_body(dst_hbm, deg_hbm, dst_v, ones_v, zero_v, acc_s, bounce_v):
    cid = lax.axis_index("c")
    sid = lax.axis_index("s")
    wid = cid * NS + sid

    _zero_fill(zero_v, RPT)
    pltpu.sync_copy(zero_v, acc_s.at[pl.ds(sid * RPT, RPT)])
    plsc.subcore_barrier()

    pltpu.sync_copy(dst_hbm.at[wid], dst_v)
    for i in range(0, B1, L):
        ones_v[pl.ds(i, L)] = jnp.ones((L,), jnp.float32)

    def chunk(j, carry):
        pltpu.sync_copy(ones_v, acc_s.at[dst_v.at[j]], add=True)
        return carry

    lax.fori_loop(0, C1, chunk, 0)
    plsc.subcore_barrier()

    pltpu.sync_copy(acc_s.at[pl.ds(sid * RPT, RPT)], bounce_v)
    pltpu.sync_copy(bounce_v, deg_hbm.at[pl.ds(cid * NP + sid * RPT, RPT)])


def _deg_partials(dst_r):
    mesh = plsc.VectorSubcoreMesh(core_axis_name="c", subcore_axis_name="s")
    return pl.kernel(
        _deg_body,
        out_type=jax.ShapeDtypeStruct((NC * NP,), jnp.float32),
        mesh=mesh,
        compiler_params=pltpu.CompilerParams(needs_layout_passes=False),
        scratch_types=[
            pltpu.VMEM((C1, B1), jnp.int32),
            pltpu.VMEM((B1,), jnp.float32),
            pltpu.VMEM((RPT,), jnp.float32),
            pltpu.VMEM_SHARED((NP,), jnp.float32),
            pltpu.VMEM((RPT,), jnp.float32),
        ],
    )(dst_r)


def _spmm_body(xs_hbm, src_hbm, dst_hbm, zeros_hbm, out_hbm,
               src_v, dst_v, r2, acc_s, sem):
    cid = lax.axis_index("c")
    sid = lax.axis_index("s")
    wid = cid * NS + sid

    pltpu.sync_copy(zeros_hbm, r2)
    for t in range(RPT // B1):
        pltpu.sync_copy(r2, acc_s.at[pl.ds(sid * RPT + t * B1, B1)])
    plsc.subcore_barrier()

    pltpu.sync_copy(src_hbm.at[wid], src_v)
    pltpu.sync_copy(dst_hbm.at[wid], dst_v)

    def chunk(j, carry):
        pltpu.async_copy(xs_hbm.at[src_v.at[j]], r2, sem).wait()
        pltpu.sync_copy(r2, acc_s.at[dst_v.at[j]], add=True)
        return carry

    lax.fori_loop(0, C1, chunk, 0)
    plsc.subcore_barrier()

    for t in range(RPT // B1):
        base = sid * RPT + t * B1
        pltpu.sync_copy(acc_s.at[pl.ds(base, B1)], r2)
        pltpu.sync_copy(r2, out_hbm.at[pl.ds(cid * NP + base, B1)])


def _spmm_partials(xs, src_r, dst_r, zeros_bd):
    mesh = plsc.VectorSubcoreMesh(core_axis_name="c", subcore_axis_name="s")
    return pl.kernel(
        _spmm_body,
        out_type=jax.ShapeDtypeStruct((NC * NP, D), jnp.float32),
        mesh=mesh,
        compiler_params=pltpu.CompilerParams(needs_layout_passes=False),
        scratch_types=[
            pltpu.VMEM((C1, B1), jnp.int32),
            pltpu.VMEM((C1, B1), jnp.int32),
            pltpu.VMEM((B1, D), jnp.float32),
            pltpu.VMEM_SHARED((NP, D), jnp.float32),
            pltpu.SemaphoreType.DMA,
        ],
    )(xs, src_r, dst_r, zeros_bd)


def _l2_body(gsx_hbm, gsy_hbm, src2_hbm, dst2_hbm, t2x_hbm, t2y_hbm,
             dinv_hbm, pa_hbm, pb_hbm,
             embx_hbm, emby_hbm, ss_hbm,
             src2_v, dst2_v, rx_v, ry_v, zero_v,
             accx_s, accy_s, embx_s, emby_s,
             ax_v, ay_v, tx_v, ty_v, dv_v,
             exf_v, eyf_v, pj_v, qj_v, ss_v, semx, semy):
    cid = lax.axis_index("c")
    sid = lax.axis_index("s")
    wid = cid * NS + sid

    _zero_fill(zero_v, RPT)
    pltpu.sync_copy(zero_v, accx_s.at[pl.ds(sid * RPT, RPT)])
    pltpu.sync_copy(zero_v, accy_s.at[pl.ds(sid * RPT, RPT)])
    plsc.subcore_barrier()

    pltpu.sync_copy(src2_hbm.at[sid], src2_v)
    pltpu.sync_copy(dst2_hbm.at[sid], dst2_v)

    def chunk(j, carry):
        cx = pltpu.async_copy(gsx_hbm.at[src2_v.at[j]], rx_v, semx)
        cy = pltpu.async_copy(gsy_hbm.at[src2_v.at[j]], ry_v, semy)
        cx.wait()
        cy.wait()
        pltpu.sync_copy(rx_v, accx_s.at[dst2_v.at[j]], add=True)
        pltpu.sync_copy(ry_v, accy_s.at[dst2_v.at[j]], add=True)
        return carry

    lax.fori_loop(0, C2, chunk, 0)
    plsc.subcore_barrier()

    rbase = sid * RPT
    pltpu.sync_copy(accx_s.at[pl.ds(rbase, RPT)], ax_v)
    pltpu.sync_copy(accy_s.at[pl.ds(rbase, RPT)], ay_v)
    pltpu.sync_copy(t2x_hbm.at[pl.ds(rbase, RPT)], tx_v)
    pltpu.sync_copy(t2y_hbm.at[pl.ds(rbase, RPT)], ty_v)
    pltpu.sync_copy(dinv_hbm.at[pl.ds(rbase, RPT)], dv_v)

    def emb_row(k, carry):
        s = pl.ds(k * L, L)
        dv = dv_v[s]
        ax_v[s] = dv * ax_v[s] + tx_v[s]
        ay_v[s] = dv * ay_v[s] + ty_v[s]
        return carry

    lax.fori_loop(0, RPT // L, emb_row, 0)
    pltpu.sync_copy(ax_v, embx_s.at[pl.ds(rbase, RPT)])
    pltpu.sync_copy(ay_v, emby_s.at[pl.ds(rbase, RPT)])

    @pl.when(cid == 0)
    def _():
        pltpu.sync_copy(ax_v, embx_hbm.at[pl.ds(rbase, RPT)])
        pltpu.sync_copy(ay_v, emby_hbm.at[pl.ds(rbase, RPT)])

    plsc.subcore_barrier()

    pltpu.sync_copy(embx_s, exf_v)
    pltpu.sync_copy(emby_s, eyf_v)
    pbase = wid * (CP * B1)

    def pchunk(j, carry):
        pltpu.sync_copy(pa_hbm.at[pl.ds(pbase + j * B1, B1)], pj_v)
        pltpu.sync_copy(pb_hbm.at[pl.ds(pbase + j * B1, B1)], qj_v)
        for m in range(B1 // L):
            s = pl.ds(m * L, L)
            a = pj_v[s]
            b = qj_v[s]
            dx = plsc.load_gather(exf_v, [a]) - plsc.load_gather(exf_v, [b])
            dy = plsc.load_gather(eyf_v, [a]) - plsc.load_gather(eyf_v, [b])
            ss_v[s] = dx * dx + dy * dy
        pltpu.sync_copy(ss_v, ss_hbm.at[pl.ds(pbase + j * B1, B1)])
        return carry

    lax.fori_loop(0, CP, pchunk, 0)


def _layer2_and_dist(gsx, gsy, src2_r, dst2_r, t2x, t2y, dinv, pa, pb):
    mesh = plsc.VectorSubcoreMesh(core_axis_name="c", subcore_axis_name="s")
    return pl.kernel(
        _l2_body,
        out_type=(
            jax.ShapeDtypeStruct((NP,), jnp.float32),
            jax.ShapeDtypeStruct((NP,), jnp.float32),
            jax.ShapeDtypeStruct((2 * E,), jnp.float32),
        ),
        mesh=mesh,
        compiler_params=pltpu.CompilerParams(needs_layout_passes=False),
        scratch_types=[
            pltpu.VMEM((C2, B1), jnp.int32),
            pltpu.VMEM((C2, B1), jnp.int32),
            pltpu.VMEM((B1,), jnp.float32),
            pltpu.VMEM((B1,), jnp.float32),
            pltpu.VMEM((RPT,), jnp.float32),
            pltpu.VMEM_SHARED((NP,), jnp.float32),
            pltpu.VMEM_SHARED((NP,), jnp.float32),
            pltpu.VMEM_SHARED((NP,), jnp.float32),
            pltpu.VMEM_SHARED((NP,), jnp.float32),
            pltpu.VMEM((RPT,), jnp.float32),
            pltpu.VMEM((RPT,), jnp.float32),
            pltpu.VMEM((RPT,), jnp.float32),
            pltpu.VMEM((RPT,), jnp.float32),
            pltpu.VMEM((RPT,), jnp.float32),
            pltpu.VMEM((NP,), jnp.float32),
            pltpu.VMEM((NP,), jnp.float32),
            pltpu.VMEM((B1,), jnp.int32),
            pltpu.VMEM((B1,), jnp.int32),
            pltpu.VMEM((B1,), jnp.float32),
            pltpu.SemaphoreType.DMA,
            pltpu.SemaphoreType.DMA,
        ],
    )(gsx, gsy, src2_r, dst2_r, t2x, t2y, dinv, pa, pb)


def kernel(features, edge_index, row_neg, col_neg, W1, b1, W2, b2):
    src = edge_index[0]
    dst = edge_index[1]

    dst_r = dst.reshape(NW, C1, B1)
    src_r = src.reshape(NW, C1, B1)
    src2_r = src.reshape(NS, C2, B1)
    dst2_r = dst.reshape(NS, C2, B1)
    pa = jnp.concatenate([src, row_neg], axis=0)
    pb = jnp.concatenate([dst, col_neg], axis=0)

    x_pad = jnp.pad(features, ((0, NP - N), (0, 0)))

    deg_p = _deg_partials(dst_r)
    deg = deg_p[:NP] + deg_p[NP:] + 1.0
    dinv = lax.rsqrt(deg)
    dinv2 = dinv * dinv
    xs = dinv[:, None] * x_pad

    acc_p = _spmm_partials(xs, src_r, dst_r, jnp.zeros((B1, D), jnp.float32))
    out1 = dinv[:, None] * (acc_p[:NP] + acc_p[NP:]) + dinv2[:, None] * x_pad
    h = jax.nn.relu(out1 @ W1 + b1)
    g = h @ W2
    gs = dinv[:, None] * g
    t2 = dinv2[:, None] * g + b2

    embx, emby, ss = _layer2_and_dist(
        gs[:, 0], gs[:, 1], src2_r, dst2_r, t2[:, 0], t2[:, 1], dinv, pa, pb)
    emb = jnp.stack([embx[:N], emby[:N]], axis=1)

    q = 1.0 / (1.0 + ALPHA * jnp.power(ss + 1e-12, BETA))
    return (emb, q)
